# baseline (speedup 1.0000x reference)
import functools

import jax
import jax.numpy as jnp
from jax import lax
from jax.experimental import pallas as pl
from jax.experimental.pallas import tpu as pltpu
from jax.experimental.pallas import tpu_sc as plsc

NUM_CORES = 2
NUM_SUBCORES = 16
NW = NUM_CORES * NUM_SUBCORES

B = 16384
D = 64
DV = 128
HIDDEN = 128
CHUNK = 128
B_PER_W = B // NW
CH_PER_W = B_PER_W // CHUNK
STAGE_CH = 2
N_STAGE = CH_PER_W // STAGE_CH


def _sc_gather(ut2, it2, uidx, iidx):
    mesh = plsc.VectorSubcoreMesh(core_axis_name="c", subcore_axis_name="s")

    @functools.partial(
        pl.kernel,
        out_type=(
            jax.ShapeDtypeStruct((B, DV), jnp.float32),
            jax.ShapeDtypeStruct((B, DV), jnp.float32),
        ),
        mesh=mesh,
        scratch_types=[
            pltpu.VMEM((B_PER_W,), jnp.int32),
            pltpu.VMEM((B_PER_W,), jnp.int32),
            pltpu.VMEM((STAGE_CH * CHUNK, DV), jnp.float32),
            pltpu.VMEM((STAGE_CH * CHUNK, DV), jnp.float32),
            pltpu.SemaphoreType.DMA,
            pltpu.SemaphoreType.DMA,
        ],
    )
    def gather_kernel(ut_hbm, it_hbm, uix_hbm, iix_hbm, uout_hbm, iout_hbm,
                      uix_v, iix_v, urows, irows, usem, isem):
        wid = lax.axis_index("s") * NUM_CORES + lax.axis_index("c")
        base = wid * B_PER_W
        pltpu.sync_copy(uix_hbm.at[pl.ds(base, B_PER_W)], uix_v)
        pltpu.sync_copy(iix_hbm.at[pl.ds(base, B_PER_W)], iix_v)
        for s in range(N_STAGE):
            copies = []
            for j in range(STAGE_CH):
                src = pl.ds((s * STAGE_CH + j) * CHUNK, CHUNK)
                dst = pl.ds(j * CHUNK, CHUNK)
                copies.append(pltpu.async_copy(
                    ut_hbm.at[uix_v.at[src]], urows.at[dst], usem))
                copies.append(pltpu.async_copy(
                    it_hbm.at[iix_v.at[src]], irows.at[dst], isem))
            for c in copies:
                c.wait()
            out = pl.ds(base + s * STAGE_CH * CHUNK, STAGE_CH * CHUNK)
            pltpu.sync_copy(urows, uout_hbm.at[out])
            pltpu.sync_copy(irows, iout_hbm.at[out])

    return gather_kernel(ut2, it2, uidx, iidx)


def _mlp_body(ue_ref, up_ref, ie_ref, ip_ref,
              wu1, bu1, wu2, bu2, wi1, bi1, wi2, bi2,
              uo_ref, io_ref):
    up = up_ref[...]
    u = ue_ref[:, :D] * (1.0 - up) + ue_ref[:, D:] * up
    hu = jnp.maximum(
        jnp.dot(u, wu1[...], preferred_element_type=jnp.float32) + bu1[...], 0.0)
    uo_ref[...] = jnp.dot(hu, wu2[...], preferred_element_type=jnp.float32) + bu2[...]
    ip = ip_ref[...]
    it = ie_ref[:, :D] * (1.0 - ip) + ie_ref[:, D:] * ip
    hi = jnp.maximum(
        jnp.dot(it, wi1[...], preferred_element_type=jnp.float32) + bi1[...], 0.0)
    io_ref[...] = jnp.dot(hi, wi2[...], preferred_element_type=jnp.float32) + bi2[...]


def _tc_mlp(ue, up, ie, ip, Wu1, bu1, Wu2, bu2, Wi1, bi1, Wi2, bi2):
    BM = 2048
    grid = (B // BM,)
    row_spec = pl.BlockSpec((BM, DV), lambda i: (i, 0))
    par_spec = pl.BlockSpec((BM, 1), lambda i: (i, 0))
    out_spec = pl.BlockSpec((BM, D), lambda i: (i, 0))
    hid_w = pl.BlockSpec((D, HIDDEN), lambda i: (0, 0))
    out_w = pl.BlockSpec((HIDDEN, D), lambda i: (0, 0))
    hid_b = pl.BlockSpec((1, HIDDEN), lambda i: (0, 0))
    out_b = pl.BlockSpec((1, D), lambda i: (0, 0))
    return pl.pallas_call(
        _mlp_body,
        grid=grid,
        in_specs=[row_spec, par_spec, row_spec, par_spec,
                  hid_w, hid_b, out_w, out_b,
                  hid_w, hid_b, out_w, out_b],
        out_specs=[out_spec, out_spec],
        out_shape=[
            jax.ShapeDtypeStruct((B, D), jnp.float32),
            jax.ShapeDtypeStruct((B, D), jnp.float32),
        ],
    )(ue, up, ie, ip,
      Wu1, bu1.reshape(1, HIDDEN), Wu2, bu2.reshape(1, D),
      Wi1, bi1.reshape(1, HIDDEN), Wi2, bi2.reshape(1, D))


def kernel(user_input, item_input, user_table, item_table,
           Wu1, bu1, Wu2, bu2, Wi1, bi1, Wi2, bi2):
    ut2 = user_table.reshape(500_000, DV)
    it2 = item_table.reshape(500_000, DV)
    uidx = jnp.right_shift(user_input, 1)
    iidx = jnp.right_shift(item_input, 1)
    up = jnp.bitwise_and(user_input, 1).astype(jnp.float32).reshape(B, 1)
    ip = jnp.bitwise_and(item_input, 1).astype(jnp.float32).reshape(B, 1)
    ue, ie = _sc_gather(ut2, it2, uidx, iidx)
    uo, io = _tc_mlp(ue, up, ie, ip, Wu1, bu1, Wu2, bu2, Wi1, bi1, Wi2, bi2)
    return (uo, io)


# speedup vs baseline: 1.9839x; 1.9839x over previous
import functools

import jax
import jax.numpy as jnp
from jax import lax
from jax.experimental import pallas as pl
from jax.experimental.pallas import tpu as pltpu
from jax.experimental.pallas import tpu_sc as plsc

NUM_CORES = 2
NUM_SUBCORES = 16
NW = NUM_CORES * NUM_SUBCORES

B = 16384
D = 64
DV = 128
N_ROWS = 1_000_000
NV = N_ROWS // 2
HIDDEN = 128
CHUNK = 128
B_PER_W = B // NW
CH_PER_W = B_PER_W // CHUNK

TBM = 8192
TBH = TBM // 2
NBLK = (N_ROWS + TBM - 1) // TBM
NV2 = NBLK * TBH


def _transpose_body(src_ref, dst_ref):
    x = src_ref[...]
    dst_ref[:, :D] = x[:, :TBH].T
    dst_ref[:, D:] = x[:, TBH:].T


def _tc_transpose(tT):
    return pl.pallas_call(
        _transpose_body,
        grid=(NBLK,),
        in_specs=[pl.BlockSpec((D, TBM), lambda i: (0, i))],
        out_specs=pl.BlockSpec((TBH, DV), lambda i: (i, 0)),
        out_shape=jax.ShapeDtypeStruct((NV2, DV), jnp.float32),
    )(tT)


def _sc_gather(t2, vidx):
    mesh = plsc.VectorSubcoreMesh(core_axis_name="c", subcore_axis_name="s")

    @functools.partial(
        pl.kernel,
        out_type=jax.ShapeDtypeStruct((B, DV), jnp.float32),
        mesh=mesh,
        compiler_params=pltpu.CompilerParams(use_tc_tiling_on_sc=False),
        scratch_types=[
            pltpu.VMEM((B_PER_W,), jnp.int32),
            pltpu.VMEM((B_PER_W, DV), jnp.float32),
            pltpu.SemaphoreType.DMA,
        ],
    )
    def gather_kernel(t2_hbm, ix_hbm, out_hbm, ix_v, rows, sem):
        wid = lax.axis_index("s") * NUM_CORES + lax.axis_index("c")
        base = wid * B_PER_W
        pltpu.sync_copy(ix_hbm.at[pl.ds(base, B_PER_W)], ix_v)
        copies = []
        for j in range(CH_PER_W):
            sl = pl.ds(j * CHUNK, CHUNK)
            copies.append(
                pltpu.async_copy(t2_hbm.at[ix_v.at[sl]], rows.at[sl], sem))
        for c in copies:
            c.wait()
        pltpu.sync_copy(rows, out_hbm.at[pl.ds(base, B_PER_W)])

    return gather_kernel(t2, vidx)


def _mlp_body(ue_ref, up_ref, ie_ref, ip_ref,
              wu1, bu1, wu2, bu2, wi1, bi1, wi2, bi2,
              uo_ref, io_ref):
    up = up_ref[...]
    u = ue_ref[:, :D] * (1.0 - up) + ue_ref[:, D:] * up
    hu = jnp.maximum(
        jnp.dot(u, wu1[...], preferred_element_type=jnp.float32) + bu1[...], 0.0)
    uo_ref[...] = jnp.dot(hu, wu2[...], preferred_element_type=jnp.float32) + bu2[...]
    ip = ip_ref[...]
    it = ie_ref[:, :D] * (1.0 - ip) + ie_ref[:, D:] * ip
    hi = jnp.maximum(
        jnp.dot(it, wi1[...], preferred_element_type=jnp.float32) + bi1[...], 0.0)
    io_ref[...] = jnp.dot(hi, wi2[...], preferred_element_type=jnp.float32) + bi2[...]


def _tc_mlp(ue, up, ie, ip, Wu1, bu1, Wu2, bu2, Wi1, bi1, Wi2, bi2):
    BM = 2048
    grid = (B // BM,)
    row_spec = pl.BlockSpec((BM, DV), lambda i: (i, 0))
    par_spec = pl.BlockSpec((BM, 1), lambda i: (i, 0))
    out_spec = pl.BlockSpec((BM, D), lambda i: (i, 0))
    hid_w = pl.BlockSpec((D, HIDDEN), lambda i: (0, 0))
    out_w = pl.BlockSpec((HIDDEN, D), lambda i: (0, 0))
    hid_b = pl.BlockSpec((1, HIDDEN), lambda i: (0, 0))
    out_b = pl.BlockSpec((1, D), lambda i: (0, 0))
    return pl.pallas_call(
        _mlp_body,
        grid=grid,
        in_specs=[row_spec, par_spec, row_spec, par_spec,
                  hid_w, hid_b, out_w, out_b,
                  hid_w, hid_b, out_w, out_b],
        out_specs=[out_spec, out_spec],
        out_shape=[
            jax.ShapeDtypeStruct((B, D), jnp.float32),
            jax.ShapeDtypeStruct((B, D), jnp.float32),
        ],
    )(ue, up, ie, ip,
      Wu1, bu1.reshape(1, HIDDEN), Wu2, bu2.reshape(1, D),
      Wi1, bi1.reshape(1, HIDDEN), Wi2, bi2.reshape(1, D))


def kernel(user_input, item_input, user_table, item_table,
           Wu1, bu1, Wu2, bu2, Wi1, bi1, Wi2, bi2):
    def vmap_idx(r):
        return jnp.bitwise_or(
            jnp.left_shift(jnp.right_shift(r, 13), 12),
            jnp.bitwise_and(r, 4095))

    def half_bit(r):
        return jnp.bitwise_and(jnp.right_shift(r, 12), 1)

    uvidx = vmap_idx(user_input)
    ividx = vmap_idx(item_input)
    up = half_bit(user_input).astype(jnp.float32).reshape(B, 1)
    ip = half_bit(item_input).astype(jnp.float32).reshape(B, 1)
    ut2 = _tc_transpose(user_table.T)
    ue = _sc_gather(ut2, uvidx)
    it2 = _tc_transpose(item_table.T)
    ie = _sc_gather(it2, ividx)
    uo, io = _tc_mlp(ue, up, ie, ip, Wu1, bu1, Wu2, bu2, Wi1, bi1, Wi2, bi2)
    return (uo, io)
